# baseline (speedup 1.0000x reference)
import functools

import jax
import jax.numpy as jnp
from jax import lax
from jax.experimental import pallas as pl
from jax.experimental.pallas import tpu as pltpu
from jax.experimental.pallas import tpu_sc as plsc

_B = 16384
_D = 32
_NC = 2
_NS = 16
_NW = _NC * _NS
_BPW = _B // _NW
_IC = 128
_NCHUNK = _BPW // _IC
_L = 16
_W = 128
_G = _IC // _L


def _sc_body(heads_hbm, tails_hbm, rels_hbm, ent_hbm, rel_hbm, out_hbm,
             hidx, tidx, ridx, hbig, tbig, rbig, e1, e2, r, out_v,
             sem0, sem1):
    wid = lax.axis_index("s") * _NC + lax.axis_index("c")
    row0 = wid * _NCHUNK
    pltpu.sync_copy(heads_hbm.at[pl.ds(row0, _NCHUNK)], hidx)
    pltpu.sync_copy(tails_hbm.at[pl.ds(row0, _NCHUNK)], tidx)
    pltpu.sync_copy(rels_hbm.at[pl.ds(row0, _NCHUNK)], ridx)

    for j in range(_NCHUNK):
        for v in range(_G):
            sl = pl.ds(v * _L, _L)
            hbig[j, sl] = lax.shift_right_logical(hidx[j, sl], 2)
            tbig[j, sl] = lax.shift_right_logical(tidx[j, sl], 2)
            rbig[j, sl] = lax.shift_right_logical(ridx[j, sl], 2)

    sems = (sem0, sem1)
    iota = lax.iota(jnp.int32, _L)
    three = jnp.full((_L,), 3, jnp.int32)
    pending = {}

    def fire(ci):
        buf = ci % 2
        dst = pl.ds(buf * _IC, _IC)
        pending[ci] = [
            pltpu.async_copy(ent_hbm.at[hbig.at[ci]], e1.at[dst], sems[buf]),
            pltpu.async_copy(ent_hbm.at[tbig.at[ci]], e2.at[dst], sems[buf]),
            pltpu.async_copy(rel_hbm.at[rbig.at[ci]], r.at[dst], sems[buf]),
        ]

    fire(0)
    for ci in range(_NCHUNK):
        if ci + 1 < _NCHUNK:
            fire(ci + 1)
        for c in pending.pop(ci):
            c.wait()
        buf = ci % 2

        def group(g, carry):
            sl = pl.ds(pl.multiple_of(g * _L, _L), _L)
            hq = lax.shift_left(hidx[ci, sl] & three, 5)
            tq = lax.shift_left(tidx[ci, sl] & three, 5)
            rq = lax.shift_left(ridx[ci, sl] & three, 5)
            rows = buf * _IC + pl.multiple_of(g * _L, _L) + iota
            acc = jnp.zeros((_L,), jnp.float32)
            for d in range(_D):
                dd = jnp.full((_L,), d, jnp.int32)
                a = plsc.load_gather(e1, [rows, hq + dd])
                b = plsc.load_gather(r, [rows, rq + dd])
                c = plsc.load_gather(e2, [rows, tq + dd])
                acc = acc + a * b * c
            pred = 1.0 / (1.0 + jnp.exp(-acc))
            out_v[pl.ds(ci * _IC + pl.multiple_of(g * _L, _L), _L)] = pred
            return carry

        lax.fori_loop(0, _G, group, 0)

    pltpu.sync_copy(out_v, out_hbm.at[pl.ds(wid * _BPW, _BPW)])


_sc_call = functools.partial(
    pl.kernel,
    out_type=jax.ShapeDtypeStruct((_B,), jnp.float32),
    mesh=plsc.VectorSubcoreMesh(core_axis_name="c", subcore_axis_name="s"),
    compiler_params=pltpu.CompilerParams(
        use_tc_tiling_on_sc=False, needs_layout_passes=False
    ),
    scratch_types=[
        pltpu.VMEM((_NCHUNK, _IC), jnp.int32),
        pltpu.VMEM((_NCHUNK, _IC), jnp.int32),
        pltpu.VMEM((_NCHUNK, _IC), jnp.int32),
        pltpu.VMEM((_NCHUNK, _IC), jnp.int32),
        pltpu.VMEM((_NCHUNK, _IC), jnp.int32),
        pltpu.VMEM((_NCHUNK, _IC), jnp.int32),
        pltpu.VMEM((2 * _IC, _W), jnp.float32),
        pltpu.VMEM((2 * _IC, _W), jnp.float32),
        pltpu.VMEM((2 * _IC, _W), jnp.float32),
        pltpu.VMEM((_BPW,), jnp.float32),
        pltpu.SemaphoreType.DMA,
        pltpu.SemaphoreType.DMA,
    ],
)(_sc_body)


@jax.jit
def kernel(heads, tails, relations, entity_embedding, relation_embedding):
    h2 = heads.reshape(_B // _IC, _IC)
    t2 = tails.reshape(_B // _IC, _IC)
    r2 = relations.reshape(_B // _IC, _IC)
    ent = entity_embedding.reshape(-1, _W)
    rel = relation_embedding.reshape(-1, _W)
    return _sc_call(h2, t2, r2, ent, rel)
